# baseline (speedup 1.0000x reference)
import jax
import jax.numpy as jnp
from jax.experimental import pallas as pl
from jax.experimental.pallas import tpu as pltpu

_M = 1048576
_N = 32
_CH = _M // _N
_NB = 8
_H = 4


def _read(x_hbm, buf, rsem, c):
    return pltpu.make_async_copy(
        x_hbm.at[:, pl.ds(c * _CH, _CH)], buf.at[c % _NB], rsem.at[c % _NB])


def _write(o_hbm, buf, wsem, c):
    return pltpu.make_async_copy(
        buf.at[c % _NB], o_hbm.at[:, pl.ds(c * _CH, _CH)], wsem.at[c % _NB])


def _body(x_hbm, a_ref, index_ref, o_hbm, buf, rsem, wsem):
    for c in range(_H):
        _read(x_hbm, buf, rsem, c).start()

    lanes = jax.lax.broadcasted_iota(jnp.int32, (1, 128), 1)
    for i in range(_N):
        s = i % _NB
        _read(x_hbm, buf, rsem, i).wait()

        for k in range(3):
            idx = index_ref[k]
            rel = idx - i * _CH
            in_chunk = jnp.logical_and(idx >= i * _CH, idx < (i + 1) * _CH)

            @pl.when(in_chunk)
            def _():
                win = pl.multiple_of((rel // 128) * 128, 128)
                lane = rel - (rel // 128) * 128
                hit = lanes == lane
                for j in range(3):
                    sub = buf[s, j:j + 1, pl.ds(win, 128)]
                    buf[s, j:j + 1, pl.ds(win, 128)] = (
                        sub + jnp.where(hit, a_ref[k, j], 0.0))

        _write(o_hbm, buf, wsem, i).start()

        r = i + _H
        if r < _N:
            if r >= _NB:
                _write(o_hbm, buf, wsem, r - _NB).wait()
            _read(x_hbm, buf, rsem, r).start()

    for c in range(_N - _NB, _N):
        _write(o_hbm, buf, wsem, c).wait()


def kernel(x, a, index):
    out = pl.pallas_call(
        _body,
        in_specs=[
            pl.BlockSpec(memory_space=pl.ANY),
            pl.BlockSpec(memory_space=pltpu.SMEM),
            pl.BlockSpec(memory_space=pltpu.SMEM),
        ],
        out_specs=pl.BlockSpec(memory_space=pl.ANY),
        out_shape=jax.ShapeDtypeStruct((3, _M), jnp.float32),
        scratch_shapes=[
            pltpu.VMEM((_NB, 3, _CH), jnp.float32),
            pltpu.SemaphoreType.DMA((_NB,)),
            pltpu.SemaphoreType.DMA((_NB,)),
        ],
    )(x, a, index.astype(jnp.int32))
    return jnp.transpose(out, (1, 0))


# speedup vs baseline: 1.3254x; 1.3254x over previous
import jax
import jax.numpy as jnp
from jax.experimental import pallas as pl
from jax.experimental.pallas import tpu as pltpu

_M = 1048576
_K = 8
_CH = _M // _K


def _win(index_ref, k):
    return pl.multiple_of((index_ref[k] // 128) * 128, 128)


def _body(x_ref, a_ref, index_ref, o_hbm, tbuf, csem, wsem):
    for c in range(_K):
        pltpu.make_async_copy(
            x_ref.at[:, pl.ds(c * _CH, _CH)],
            o_hbm.at[:, pl.ds(c * _CH, _CH)],
            csem.at[c]).start()

    lanes = jax.lax.broadcasted_iota(jnp.int32, (1, 128), 1)
    for k in range(3):
        win = _win(index_ref, k)
        rows = []
        for j in range(3):
            r = x_ref[j:j + 1, pl.ds(win, 128)]
            for k2 in range(3):
                rel = index_ref[k2] - win
                r = r + jnp.where(lanes == rel, a_ref[k2, j], 0.0)
            rows.append(r)
        tbuf[k] = jnp.concatenate(rows, axis=0)

    for c in range(_K):
        pltpu.make_async_copy(
            x_ref.at[:, pl.ds(c * _CH, _CH)],
            o_hbm.at[:, pl.ds(c * _CH, _CH)],
            csem.at[c]).wait()
        for k in range(3):
            win = _win(index_ref, k)

            @pl.when(win // _CH == c)
            def _():
                pltpu.make_async_copy(
                    tbuf.at[k],
                    o_hbm.at[:, pl.ds(win, 128)],
                    wsem.at[k]).start()

    for k in range(3):
        pltpu.make_async_copy(
            tbuf.at[k],
            o_hbm.at[:, pl.ds(_win(index_ref, k), 128)],
            wsem.at[k]).wait()


def kernel(x, a, index):
    out = pl.pallas_call(
        _body,
        in_specs=[
            pl.BlockSpec(memory_space=pltpu.VMEM),
            pl.BlockSpec(memory_space=pltpu.SMEM),
            pl.BlockSpec(memory_space=pltpu.SMEM),
        ],
        out_specs=pl.BlockSpec(memory_space=pl.ANY),
        out_shape=jax.ShapeDtypeStruct((3, _M), jnp.float32),
        scratch_shapes=[
            pltpu.VMEM((3, 3, 128), jnp.float32),
            pltpu.SemaphoreType.DMA((_K,)),
            pltpu.SemaphoreType.DMA((3,)),
        ],
    )(x, a, index.astype(jnp.int32))
    return jnp.transpose(out, (1, 0))
